# baseline (speedup 1.0000x reference)
import functools

import jax
import jax.numpy as jnp
from jax import lax
from jax.experimental import pallas as pl
from jax.experimental.pallas import tpu as pltpu
from jax.experimental.pallas import tpu_sc as plsc

VOCAB = 1000
N_EMBD = 64
EMBD_PAD = 128
NUM_WORKERS = 32
CHUNK = 128
BTILE = 4096


@functools.lru_cache(maxsize=None)
def _make_emb_gather(T, B):
    n_tok = T * B
    assert n_tok % (NUM_WORKERS * CHUNK) == 0
    tok_per_w = n_tok // NUM_WORKERS
    n_chunks = tok_per_w // CHUNK
    assert B % tok_per_w == 0
    mesh = plsc.VectorSubcoreMesh(core_axis_name="c", subcore_axis_name="s")

    assert n_chunks % 2 == 0
    n_iters = n_chunks // 2

    @functools.partial(
        pl.kernel,
        out_type=jax.ShapeDtypeStruct((T, B, EMBD_PAD), jnp.float32),
        mesh=mesh,
        scratch_types=[
            pltpu.VMEM((n_chunks, CHUNK), jnp.int32),
            pltpu.VMEM((CHUNK, EMBD_PAD), jnp.float32),
            pltpu.VMEM((CHUNK, EMBD_PAD), jnp.float32),
            pltpu.SemaphoreType.DMA,
            pltpu.SemaphoreType.DMA,
            pltpu.SemaphoreType.DMA,
            pltpu.SemaphoreType.DMA,
        ],
        compiler_params=pltpu.CompilerParams(needs_layout_passes=False),
    )
    def gather(table_hbm, idx_hbm, out_hbm, idx_v, rows_a, rows_b,
               sem_ga, sem_gb, sem_wa, sem_wb):
        wid = lax.axis_index("s") * 2 + lax.axis_index("c")
        t = wid // (B // tok_per_w)
        b_base = (wid * tok_per_w) % B

        def start_gather(c, rows_v, sem):
            pltpu.async_copy(table_hbm.at[idx_v.at[c]], rows_v, sem)

        def wait_gather(c, rows_v, sem):
            pltpu.make_async_copy(table_hbm.at[idx_v.at[c]], rows_v, sem).wait()

        def start_write(c, rows_v, sem):
            pltpu.async_copy(
                rows_v, out_hbm.at[t, pl.ds(b_base + c * CHUNK, CHUNK)], sem)

        def wait_write(c, rows_v, sem):
            pltpu.make_async_copy(
                rows_v, out_hbm.at[t, pl.ds(b_base + c * CHUNK, CHUNK)], sem).wait()

        pltpu.sync_copy(idx_hbm.at[wid], idx_v)
        start_gather(0, rows_a, sem_ga)

        def body(g, carry):
            c0 = 2 * g

            @pl.when(g > 0)
            def _():
                wait_write(c0 - 1, rows_b, sem_wb)

            start_gather(c0 + 1, rows_b, sem_gb)
            wait_gather(c0, rows_a, sem_ga)
            start_write(c0, rows_a, sem_wa)

            @pl.when(g < n_iters - 1)
            def _():
                wait_write(c0, rows_a, sem_wa)
                start_gather(c0 + 2, rows_a, sem_ga)

            wait_gather(c0 + 1, rows_b, sem_gb)
            start_write(c0 + 1, rows_b, sem_wb)
            return carry

        lax.fori_loop(0, n_iters, body, 0)
        wait_write(n_chunks - 2, rows_a, sem_wa)
        wait_write(n_chunks - 1, rows_b, sem_wb)

    return gather


def _matmul_body(emb_ref, w_ref, b_ref, out_ref):
    e = emb_ref[0, :, :N_EMBD]
    s = lax.dot_general(
        w_ref[...], e, (((0,), (1,)), ((), ())),
        preferred_element_type=jnp.float32)
    out_ref[0] = s + b_ref[...]


def _matmul_body_acc(emb_ref, w_ref, b_ref, prev_ref, out_ref):
    del prev_ref
    _matmul_body(emb_ref, w_ref, b_ref, out_ref)


@functools.lru_cache(maxsize=None)
def _make_matmul(T, T_slab, t_off, B, aliased):
    assert B % BTILE == 0
    in_specs = [
        pl.BlockSpec((1, BTILE, EMBD_PAD), lambda t, bt: (t, bt, 0)),
        pl.BlockSpec((N_EMBD, VOCAB), lambda t, bt: (0, 0)),
        pl.BlockSpec((VOCAB, 1), lambda t, bt: (0, 0)),
    ]
    body = _matmul_body
    kwargs = {}
    if aliased:
        in_specs.append(pl.BlockSpec(memory_space=pl.ANY))
        body = _matmul_body_acc
        kwargs["input_output_aliases"] = {3: 0}
    return pl.pallas_call(
        body,
        grid=(T_slab, B // BTILE),
        in_specs=in_specs,
        out_specs=pl.BlockSpec((1, VOCAB, BTILE),
                               lambda t, bt: (t + t_off, 0, bt)),
        out_shape=jax.ShapeDtypeStruct((T, VOCAB, B), jnp.float32),
        **kwargs,
    )


N_SLABS = 2


def kernel(idx, tok_table, pos_table, W, b):
    B, T = idx.shape
    assert T % N_SLABS == 0
    t_slab = T // N_SLABS
    tok_pad = jnp.pad(tok_table, ((0, 0), (0, EMBD_PAD - N_EMBD)))
    idx_t = idx.T.astype(jnp.int32)
    b2 = b.reshape(VOCAB, 1)
    n_tok = t_slab * B
    embs = []
    for s in range(N_SLABS):
        idx_s = idx_t[s * t_slab:(s + 1) * t_slab].reshape(
            NUM_WORKERS, n_tok // (NUM_WORKERS * CHUNK), CHUNK)
        embs.append(_make_emb_gather(t_slab, B)(tok_pad, idx_s))
    out = _make_matmul(T, t_slab, 0, B, False)(embs[0], W, b2)
    for s in range(1, N_SLABS):
        out = _make_matmul(T, t_slab, s * t_slab, B, True)(embs[s], W, b2, out)
    return out.transpose(2, 0, 1)


# speedup vs baseline: 1.0010x; 1.0010x over previous
import functools

import jax
import jax.numpy as jnp
from jax import lax
from jax.experimental import pallas as pl
from jax.experimental.pallas import tpu as pltpu
from jax.experimental.pallas import tpu_sc as plsc

VOCAB = 1000
N_EMBD = 64
EMBD_PAD = 128
NUM_WORKERS = 32
CHUNK = 128
BTILE = 4096


@functools.lru_cache(maxsize=None)
def _make_emb_gather(T, B):
    n_tok = T * B
    assert n_tok % (NUM_WORKERS * CHUNK) == 0
    tok_per_w = n_tok // NUM_WORKERS
    n_chunks = tok_per_w // CHUNK
    assert B % tok_per_w == 0
    mesh = plsc.VectorSubcoreMesh(core_axis_name="c", subcore_axis_name="s")

    assert n_chunks % 2 == 0
    n_iters = n_chunks // 2

    @functools.partial(
        pl.kernel,
        out_type=jax.ShapeDtypeStruct((T, B, EMBD_PAD), jnp.float32),
        mesh=mesh,
        scratch_types=[
            pltpu.VMEM((n_chunks, CHUNK), jnp.int32),
            pltpu.VMEM((CHUNK, EMBD_PAD), jnp.float32),
            pltpu.VMEM((CHUNK, EMBD_PAD), jnp.float32),
            pltpu.SemaphoreType.DMA,
            pltpu.SemaphoreType.DMA,
            pltpu.SemaphoreType.DMA,
            pltpu.SemaphoreType.DMA,
        ],
        compiler_params=pltpu.CompilerParams(needs_layout_passes=False),
    )
    def gather(table_hbm, idx_hbm, out_hbm, idx_v, rows_a, rows_b,
               sem_ga, sem_gb, sem_wa, sem_wb):
        wid = lax.axis_index("s") * 2 + lax.axis_index("c")
        t = wid // (B // tok_per_w)
        b_base = (wid * tok_per_w) % B

        def start_gather(c, rows_v, sem):
            pltpu.async_copy(table_hbm.at[idx_v.at[c]], rows_v, sem)

        def wait_gather(c, rows_v, sem):
            pltpu.make_async_copy(table_hbm.at[idx_v.at[c]], rows_v, sem).wait()

        def start_write(c, rows_v, sem):
            pltpu.async_copy(
                rows_v, out_hbm.at[t, pl.ds(b_base + c * CHUNK, CHUNK)], sem)

        def wait_write(c, rows_v, sem):
            pltpu.make_async_copy(
                rows_v, out_hbm.at[t, pl.ds(b_base + c * CHUNK, CHUNK)], sem).wait()

        pltpu.sync_copy(idx_hbm.at[wid], idx_v)
        start_gather(0, rows_a, sem_ga)

        def body(g, carry):
            c0 = 2 * g

            @pl.when(g > 0)
            def _():
                wait_write(c0 - 1, rows_b, sem_wb)

            start_gather(c0 + 1, rows_b, sem_gb)
            wait_gather(c0, rows_a, sem_ga)
            start_write(c0, rows_a, sem_wa)

            @pl.when(g < n_iters - 1)
            def _():
                wait_write(c0, rows_a, sem_wa)
                start_gather(c0 + 2, rows_a, sem_ga)

            wait_gather(c0 + 1, rows_b, sem_gb)
            start_write(c0 + 1, rows_b, sem_wb)
            return carry

        lax.fori_loop(0, n_iters, body, 0)
        wait_write(n_chunks - 2, rows_a, sem_wa)
        wait_write(n_chunks - 1, rows_b, sem_wb)

    return gather


def _matmul_body(emb_ref, w_ref, b_ref, out_ref):
    e = emb_ref[0, :, :N_EMBD]
    s = lax.dot_general(
        w_ref[...], e, (((0,), (1,)), ((), ())),
        preferred_element_type=jnp.float32)
    out_ref[0] = s + b_ref[...]


@functools.lru_cache(maxsize=None)
def _make_matmul(T, B):
    assert B % BTILE == 0
    return pl.pallas_call(
        _matmul_body,
        grid=(T, B // BTILE),
        in_specs=[
            pl.BlockSpec((1, BTILE, EMBD_PAD), lambda t, bt: (t, bt, 0)),
            pl.BlockSpec((N_EMBD, VOCAB), lambda t, bt: (0, 0)),
            pl.BlockSpec((VOCAB, 1), lambda t, bt: (0, 0)),
        ],
        out_specs=pl.BlockSpec((1, VOCAB, BTILE), lambda t, bt: (t, 0, bt)),
        out_shape=jax.ShapeDtypeStruct((T, VOCAB, B), jnp.float32),
    )


def kernel(idx, tok_table, pos_table, W, b):
    B, T = idx.shape
    tok_pad = jnp.pad(tok_table, ((0, 0), (0, EMBD_PAD - N_EMBD)))
    idx_t = idx.T.reshape(NUM_WORKERS, (B * T) // (NUM_WORKERS * CHUNK), CHUNK)
    idx_t = idx_t.astype(jnp.int32)
    emb = _make_emb_gather(T, B)(tok_pad, idx_t)
    scores_t = _make_matmul(T, B)(emb, W, b.reshape(VOCAB, 1))
    return scores_t.transpose(2, 0, 1)
